# baseline (speedup 1.0000x reference)
import functools

import jax
import jax.numpy as jnp
from jax import lax
from jax.experimental import pallas as pl
from jax.experimental.pallas import tpu as pltpu
from jax.experimental.pallas import tpu_sc as plsc

_BS, _K, _S, _D = 1024, 32, 8, 128
_AGG = 0.5
_NC, _NS = 2, 16
_NW = _NC * _NS
_BPW = _BS // _NW
_V = _D // 16


def _sc_body(edge, masks, ent, selfv, sv_out, nv_out,
             ebuf, mbuf, entbuf, nvbuf, selfbuf, svbuf, accbuf,
             sem_e, sem_s, sem_o):
    wid = lax.axis_index("c") * _NS + lax.axis_index("s")
    b0 = wid * _BPW

    pltpu.sync_copy(selfv.at[pl.ds(b0, _BPW)], selfbuf)

    def start_in(j, sl):
        bb = b0 + j
        pltpu.async_copy(edge.at[pl.ds(bb * (_K * _S), _K * _S)], ebuf.at[sl], sem_e)
        pltpu.async_copy(masks.at[bb], mbuf.at[sl], sem_s)
        pltpu.async_copy(ent.at[pl.ds(bb * _K, _K)], entbuf.at[sl], sem_s)

    def wait_in(j, sl):
        bb = b0 + j
        pltpu.make_async_copy(edge.at[pl.ds(bb * (_K * _S), _K * _S)], ebuf.at[sl], sem_e).wait()
        pltpu.make_async_copy(masks.at[bb], mbuf.at[sl], sem_s).wait()
        pltpu.make_async_copy(ent.at[pl.ds(bb * _K, _K)], entbuf.at[sl], sem_s).wait()

    start_in(0, 0)

    def iter_body(i, _):
        sl = lax.rem(i, 2)
        nsl = 1 - sl
        bb = b0 + i

        @pl.when(i + 1 < _BPW)
        def _():
            start_in(i + 1, nsl)

        @pl.when(i >= 2)
        def _():
            pltpu.make_async_copy(nvbuf.at[sl], nv_out.at[pl.ds((bb - 2) * _K, _K)], sem_o).wait()

        wait_in(i, sl)

        def kk_body(kk, sv_acc):
            m16 = mbuf[sl, pl.ds(kk * 16, 16)]
            for half in range(2):
                k = kk * 2 + half
                zero = jnp.zeros((16,), jnp.float32)
                for v in range(_V):
                    accbuf[pl.ds(v * 16, 16)] = zero
                cnt = jnp.float32(0.0)
                for s in range(_S):
                    lane = half * _S + s
                    r = k * _S + s
                    m = m16[lane]
                    cnt = cnt + m

                    @pl.when(m > 0.0)
                    def _(r=r):
                        for v in range(_V):
                            plsc.addupdate(accbuf.at[pl.ds(v * 16, 16)],
                                           ebuf[sl, r, pl.ds(v * 16, 16)])
                scale = (jnp.full((16,), _AGG, jnp.float32)
                         / jnp.maximum(lax.broadcast(cnt, (16,)), 1.0))
                out = []
                for v in range(_V):
                    nv_v = (entbuf[sl, k, pl.ds(v * 16, 16)]
                            + scale * accbuf[pl.ds(v * 16, 16)])
                    nvbuf[sl, k, pl.ds(v * 16, 16)] = nv_v
                    out.append(sv_acc[v] + nv_v)
                sv_acc = tuple(out)
            return sv_acc

        sv0 = tuple(jnp.zeros((16,), jnp.float32) for _ in range(_V))
        sv = lax.fori_loop(0, _K // 2, kk_body, sv0)
        for v in range(_V):
            svbuf[i, pl.ds(v * 16, 16)] = (
                selfbuf[i, pl.ds(v * 16, 16)] + sv[v] * jnp.float32(_AGG / _K))

        pltpu.async_copy(nvbuf.at[sl], nv_out.at[pl.ds(bb * _K, _K)], sem_o)
        return 0

    lax.fori_loop(0, _BPW, iter_body, 0)

    for j in (_BPW - 2, _BPW - 1):
        pltpu.make_async_copy(
            nvbuf.at[lax.rem(jnp.int32(j), 2)],
            nv_out.at[pl.ds((b0 + j) * _K, _K)], sem_o).wait()

    pltpu.sync_copy(svbuf, sv_out.at[pl.ds(b0, _BPW)])


@functools.cache
def _build_sc_call():
    return functools.partial(
        pl.kernel,
        mesh=plsc.VectorSubcoreMesh(core_axis_name="c", subcore_axis_name="s"),
        out_type=[
            jax.ShapeDtypeStruct((_BS, _D), jnp.float32),
            jax.ShapeDtypeStruct((_BS * _K, _D), jnp.float32),
        ],
        scratch_types=[
            pltpu.VMEM((2, _K * _S, _D), jnp.float32),
            pltpu.VMEM((2, _K * _S), jnp.float32),
            pltpu.VMEM((2, _K, _D), jnp.float32),
            pltpu.VMEM((2, _K, _D), jnp.float32),
            pltpu.VMEM((_BPW, _D), jnp.float32),
            pltpu.VMEM((_BPW, _D), jnp.float32),
            pltpu.VMEM((_D,), jnp.float32),
            pltpu.SemaphoreType.DMA,
            pltpu.SemaphoreType.DMA,
            pltpu.SemaphoreType.DMA,
        ],
    )(_sc_body)


def kernel(self_vectors, neighbor_entity_vectors, neighbor_edge_vectors, masks, W, b):
    del W, b
    bs = self_vectors.shape[0]
    edge2 = neighbor_edge_vectors.reshape(_BS * _K * _S, _D)
    masks2 = masks.reshape(_BS, _K * _S)
    ent2 = neighbor_entity_vectors.reshape(_BS * _K, _D)
    self2 = self_vectors.reshape(_BS, _D)
    sv, nv = _build_sc_call()(edge2, masks2, ent2, self2)
    return (sv.reshape(bs, -1, _D), nv.reshape(_BS, 1, _K, _D))


# speedup vs baseline: 3.0886x; 3.0886x over previous
import functools

import jax
import jax.numpy as jnp
from jax import lax
from jax.experimental import pallas as pl
from jax.experimental.pallas import tpu as pltpu
from jax.experimental.pallas import tpu_sc as plsc

_BS, _K, _S, _D = 1024, 32, 8, 128
_AGG = 0.5
_NC, _NS = 2, 16
_NW = _NC * _NS
_BPW = _BS // _NW
_V = _D // 16


def _sc_body(edge, masks, ent, selfv, sv_out, nv_out,
             ebuf, mbuf, entbuf, nvbuf, selfbuf, svbuf,
             sem_e, sem_s, sem_o):
    wid = lax.axis_index("c") * _NS + lax.axis_index("s")
    b0 = wid * _BPW

    pltpu.sync_copy(selfv.at[pl.ds(b0, _BPW)], selfbuf)

    def start_in(j, sl):
        bb = b0 + j
        pltpu.async_copy(edge.at[pl.ds(bb * (_K * _S), _K * _S)], ebuf.at[sl], sem_e)
        pltpu.async_copy(masks.at[bb], mbuf.at[sl], sem_s)
        pltpu.async_copy(ent.at[pl.ds(bb * _K, _K)], entbuf.at[sl], sem_s)

    def wait_in(j, sl):
        bb = b0 + j
        pltpu.make_async_copy(edge.at[pl.ds(bb * (_K * _S), _K * _S)], ebuf.at[sl], sem_e).wait()
        pltpu.make_async_copy(masks.at[bb], mbuf.at[sl], sem_s).wait()
        pltpu.make_async_copy(ent.at[pl.ds(bb * _K, _K)], entbuf.at[sl], sem_s).wait()

    start_in(0, 0)

    def iter_body(i, _):
        sl = lax.rem(i, 2)
        nsl = 1 - sl
        bb = b0 + i

        @pl.when(i + 1 < _BPW)
        def _():
            start_in(i + 1, nsl)

        @pl.when(i >= 2)
        def _():
            pltpu.make_async_copy(nvbuf.at[sl], nv_out.at[pl.ds((bb - 2) * _K, _K)], sem_o).wait()

        wait_in(i, sl)

        def kk_body(kk, sv_acc):
            m16 = mbuf[sl, pl.ds(kk * 16, 16)]
            for half in range(2):
                k = kk * 2 + half
                cnt = jnp.float32(0.0)
                accs = [jnp.zeros((16,), jnp.float32)] * _V
                for s in range(_S):
                    lane = half * _S + s
                    r = k * _S + s
                    m = m16[lane]
                    cnt = cnt + m
                    mvec = lax.broadcast(m, (16,))
                    for v in range(_V):
                        accs[v] = accs[v] + mvec * ebuf[sl, r, pl.ds(v * 16, 16)]
                scale = (jnp.full((16,), _AGG, jnp.float32)
                         / jnp.maximum(lax.broadcast(cnt, (16,)), 1.0))
                out = []
                for v in range(_V):
                    nv_v = entbuf[sl, k, pl.ds(v * 16, 16)] + scale * accs[v]
                    nvbuf[sl, k, pl.ds(v * 16, 16)] = nv_v
                    out.append(sv_acc[v] + nv_v)
                sv_acc = tuple(out)
            return sv_acc

        sv0 = tuple(jnp.zeros((16,), jnp.float32) for _ in range(_V))
        sv = lax.fori_loop(0, _K // 2, kk_body, sv0)
        for v in range(_V):
            svbuf[i, pl.ds(v * 16, 16)] = (
                selfbuf[i, pl.ds(v * 16, 16)] + sv[v] * jnp.float32(_AGG / _K))

        pltpu.async_copy(nvbuf.at[sl], nv_out.at[pl.ds(bb * _K, _K)], sem_o)
        return 0

    lax.fori_loop(0, _BPW, iter_body, 0)

    for j in (_BPW - 2, _BPW - 1):
        pltpu.make_async_copy(
            nvbuf.at[lax.rem(jnp.int32(j), 2)],
            nv_out.at[pl.ds((b0 + j) * _K, _K)], sem_o).wait()

    pltpu.sync_copy(svbuf, sv_out.at[pl.ds(b0, _BPW)])


@functools.cache
def _build_sc_call():
    return functools.partial(
        pl.kernel,
        mesh=plsc.VectorSubcoreMesh(core_axis_name="c", subcore_axis_name="s"),
        out_type=[
            jax.ShapeDtypeStruct((_BS, _D), jnp.float32),
            jax.ShapeDtypeStruct((_BS * _K, _D), jnp.float32),
        ],
        scratch_types=[
            pltpu.VMEM((2, _K * _S, _D), jnp.float32),
            pltpu.VMEM((2, _K * _S), jnp.float32),
            pltpu.VMEM((2, _K, _D), jnp.float32),
            pltpu.VMEM((2, _K, _D), jnp.float32),
            pltpu.VMEM((_BPW, _D), jnp.float32),
            pltpu.VMEM((_BPW, _D), jnp.float32),
            pltpu.SemaphoreType.DMA,
            pltpu.SemaphoreType.DMA,
            pltpu.SemaphoreType.DMA,
        ],
    )(_sc_body)


def kernel(self_vectors, neighbor_entity_vectors, neighbor_edge_vectors, masks, W, b):
    del W, b
    bs = self_vectors.shape[0]
    edge2 = neighbor_edge_vectors.reshape(_BS * _K * _S, _D)
    masks2 = masks.reshape(_BS, _K * _S)
    ent2 = neighbor_entity_vectors.reshape(_BS * _K, _D)
    self2 = self_vectors.reshape(_BS, _D)
    sv, nv = _build_sc_call()(edge2, masks2, ent2, self2)
    return (sv.reshape(bs, -1, _D), nv.reshape(_BS, 1, _K, _D))
